# baseline (speedup 1.0000x reference)
import functools

import jax
import jax.numpy as jnp
from jax import lax
from jax.experimental import pallas as pl
from jax.experimental.pallas import tpu as pltpu
from jax.experimental.pallas import tpu_sc as plsc

_B = 128
_N = 32768
_L = 16
_NVEC = _N // _L
_NW = 32
_ROWS_PER_W = _B // _NW
_BISECT_ITERS = 22


def _row_sparsemax(row_v, cand_v):
    iota = lax.iota(jnp.int32, _L)

    def filt(i, carry):
        m_run, wptr = carry
        v = row_v[pl.ds(i * _L, _L)]
        m_run = jnp.maximum(m_run, v)
        mask = v >= (m_run - 1.0)
        mi = mask.astype(jnp.int32)
        incl = plsc.cumsum(mi)
        idx = wptr + incl - mi
        plsc.store_scatter(cand_v, [idx], v, mask=mask)
        pc = plsc.all_reduce_population_count(mask)
        return m_run, wptr + pc

    m0 = row_v[pl.ds(0, _L)]
    m_run, wptr = lax.fori_loop(
        0, _NVEC, filt, (m0, jnp.zeros((_L,), jnp.int32))
    )
    m = jnp.max(m_run)
    w = jnp.max(wptr)
    plsc.store_scatter(cand_v, [w + iota], jnp.full((_L,), m - 2.0))
    nvec = w // _L + 1

    def bisect(_, carry):
        lo, hi = carry
        t = 0.5 * (lo + hi)

        def acc_fn(j, acc):
            v = cand_v[pl.ds(j * _L, _L)]
            return acc + jnp.maximum(v - t, 0.0)

        acc = lax.fori_loop(0, nvec, acc_fn, jnp.zeros((_L,), jnp.float32))
        s = jnp.sum(acc)
        ge = s >= 1.0
        return jnp.where(ge, t, lo), jnp.where(ge, hi, t)

    lo, _ = lax.fori_loop(0, _BISECT_ITERS, bisect, (m - 1.0, m))

    def ks_fn(j, carry):
        ak, asum = carry
        v = cand_v[pl.ds(j * _L, _L)]
        above = v > lo
        ak = ak + above.astype(jnp.float32)
        asum = asum + jnp.where(above, v, 0.0)
        return ak, asum

    ak, asum = lax.fori_loop(
        0, nvec, ks_fn,
        (jnp.zeros((_L,), jnp.float32), jnp.zeros((_L,), jnp.float32)),
    )
    tau = (jnp.full((_L,), jnp.sum(asum)) - 1.0) / jnp.full((_L,), jnp.sum(ak))

    def out_fn(i, _):
        v = row_v[pl.ds(i * _L, _L)]
        row_v[pl.ds(i * _L, _L)] = jnp.maximum(v - tau, 0.0)
        return 0

    lax.fori_loop(0, _NVEC, out_fn, 0)


def kernel(inputs):
    mesh = plsc.VectorSubcoreMesh(core_axis_name="c", subcore_axis_name="s")

    @functools.partial(
        pl.kernel,
        mesh=mesh,
        out_type=jax.ShapeDtypeStruct((_B, _N), jnp.float32),
        scratch_types=[
            pltpu.VMEM((_N,), jnp.float32),
            pltpu.VMEM((_N + _L,), jnp.float32),
        ],
        compiler_params=pltpu.CompilerParams(needs_layout_passes=False),
    )
    def run(x_hbm, out_hbm, row_v, cand_v):
        wid = lax.axis_index("s") * 2 + lax.axis_index("c")

        def row_body(r, _):
            row = wid * _ROWS_PER_W + r
            pltpu.sync_copy(x_hbm.at[row], row_v)
            _row_sparsemax(row_v, cand_v)
            pltpu.sync_copy(row_v, out_hbm.at[row])
            return 0

        lax.fori_loop(0, _ROWS_PER_W, row_body, 0)

    return run(inputs)


# speedup vs baseline: 2.7116x; 2.7116x over previous
import functools

import jax
import jax.numpy as jnp
from jax import lax
from jax.experimental import pallas as pl
from jax.experimental.pallas import tpu as pltpu
from jax.experimental.pallas import tpu_sc as plsc

_B = 128
_N = 32768
_L = 16
_NVEC = _N // _L
_NW = 32
_ROWS_PER_W = _B // _NW
_BISECT_ITERS = 22
_UNROLL = 8


def _row_sparsemax(row_v, cand_v):
    iota = lax.iota(jnp.int32, _L)

    @plsc.parallel_loop(
        0, _NVEC, unroll=_UNROLL,
        carry=(jnp.full((_L,), -3e38, jnp.float32), jnp.zeros((_L,), jnp.int32)),
    )
    def filt(i, carry):
        m_run, wptr = carry
        v = row_v[pl.ds(i * _L, _L)]
        m_run = jnp.maximum(m_run, v)
        mask = v >= (m_run - 1.0)
        mi = mask.astype(jnp.int32)
        incl = plsc.cumsum(mi)
        idx = wptr + incl - mi
        plsc.store_scatter(cand_v, [idx], v, mask=mask)
        pc = plsc.all_reduce_population_count(mask)
        return m_run, wptr + pc

    m_run, wptr = filt
    m = jnp.max(m_run)
    w = wptr[0]
    plsc.store_scatter(cand_v, [w + iota], jnp.full((_L,), m - 2.0))
    nvec = w // _L + 1

    def bisect(_, carry):
        lo, hi = carry
        t = 0.5 * (lo + hi)

        def acc_fn(j, acc):
            v = cand_v[pl.ds(j * _L, _L)]
            return acc + jnp.maximum(v - t, 0.0)

        acc = lax.fori_loop(0, nvec, acc_fn, jnp.zeros((_L,), jnp.float32))
        s = jnp.sum(acc)
        ge = s >= 1.0
        return jnp.where(ge, t, lo), jnp.where(ge, hi, t)

    lo, _ = lax.fori_loop(0, _BISECT_ITERS, bisect, (m - 1.0, m))

    def ks_fn(j, carry):
        ak, asum = carry
        v = cand_v[pl.ds(j * _L, _L)]
        above = v > lo
        ak = ak + above.astype(jnp.float32)
        asum = asum + jnp.where(above, v, 0.0)
        return ak, asum

    ak, asum = lax.fori_loop(
        0, nvec, ks_fn,
        (jnp.zeros((_L,), jnp.float32), jnp.zeros((_L,), jnp.float32)),
    )
    tau = (jnp.full((_L,), jnp.sum(asum)) - 1.0) / jnp.full((_L,), jnp.sum(ak))

    @plsc.parallel_loop(0, _NVEC, unroll=_UNROLL)
    def out_loop(i):
        v = row_v[pl.ds(i * _L, _L)]
        row_v[pl.ds(i * _L, _L)] = jnp.maximum(v - tau, 0.0)


def kernel(inputs):
    mesh = plsc.VectorSubcoreMesh(core_axis_name="c", subcore_axis_name="s")

    @functools.partial(
        pl.kernel,
        mesh=mesh,
        out_type=jax.ShapeDtypeStruct((_B, _N), jnp.float32),
        scratch_types=[
            pltpu.VMEM((_N,), jnp.float32),
            pltpu.VMEM((_N + _L,), jnp.float32),
        ],
        compiler_params=pltpu.CompilerParams(needs_layout_passes=False),
    )
    def run(x_hbm, out_hbm, row_v, cand_v):
        wid = lax.axis_index("s") * 2 + lax.axis_index("c")

        def row_body(r, _):
            row = wid * _ROWS_PER_W + r
            pltpu.sync_copy(x_hbm.at[row], row_v)
            _row_sparsemax(row_v, cand_v)
            pltpu.sync_copy(row_v, out_hbm.at[row])
            return 0

        lax.fori_loop(0, _ROWS_PER_W, row_body, 0)

    return run(inputs)


# speedup vs baseline: 2.8444x; 1.0490x over previous
import functools

import jax
import jax.numpy as jnp
from jax import lax
from jax.experimental import pallas as pl
from jax.experimental.pallas import tpu as pltpu
from jax.experimental.pallas import tpu_sc as plsc

_B = 128
_N = 32768
_L = 16
_NVEC = _N // _L
_NW = 32
_ROWS_PER_W = _B // _NW
_BISECT_ITERS = 22
_UNROLL = 8


def _row_sparsemax(row_v, cand_v):
    iota = lax.iota(jnp.int32, _L)

    @plsc.parallel_loop(
        0, _NVEC, unroll=_UNROLL,
        carry=(jnp.full((_L,), -3e38, jnp.float32), jnp.zeros((_L,), jnp.int32)),
    )
    def filt(i, carry):
        m_run, wptr = carry
        v = row_v[pl.ds(i * _L, _L)]
        m_run = jnp.maximum(m_run, v)
        mask = v >= (m_run - 1.0)
        mi = mask.astype(jnp.int32)
        incl = plsc.cumsum(mi)
        idx = wptr + incl - mi
        plsc.store_scatter(cand_v, [idx], v, mask=mask)
        pc = plsc.all_reduce_population_count(mask)
        return m_run, wptr + pc

    m_run, wptr = filt
    m = jnp.max(m_run)
    w = wptr[0]
    plsc.store_scatter(cand_v, [w + iota], jnp.full((_L,), m - 2.0))
    nvec = w // _L + 1

    def bisect(_, carry):
        lo, hi = carry
        t = 0.5 * (lo + hi)

        def acc_fn(j, acc):
            v = cand_v[pl.ds(j * _L, _L)]
            return acc + jnp.maximum(v - t, 0.0)

        acc = lax.fori_loop(0, nvec, acc_fn, jnp.zeros((_L,), jnp.float32))
        s = jnp.sum(acc)
        ge = s >= 1.0
        return jnp.where(ge, t, lo), jnp.where(ge, hi, t)

    lo, _ = lax.fori_loop(0, _BISECT_ITERS, bisect, (m - 1.0, m))

    def ks_fn(j, carry):
        ak, asum = carry
        v = cand_v[pl.ds(j * _L, _L)]
        above = v > lo
        ak = ak + above.astype(jnp.float32)
        asum = asum + jnp.where(above, v, 0.0)
        return ak, asum

    ak, asum = lax.fori_loop(
        0, nvec, ks_fn,
        (jnp.zeros((_L,), jnp.float32), jnp.zeros((_L,), jnp.float32)),
    )
    tau = (jnp.full((_L,), jnp.sum(asum)) - 1.0) / jnp.full((_L,), jnp.sum(ak))

    @plsc.parallel_loop(0, _NVEC, unroll=_UNROLL)
    def out_loop(i):
        v = row_v[pl.ds(i * _L, _L)]
        row_v[pl.ds(i * _L, _L)] = jnp.maximum(v - tau, 0.0)


def kernel(inputs):
    mesh = plsc.VectorSubcoreMesh(core_axis_name="c", subcore_axis_name="s")

    @functools.partial(
        pl.kernel,
        mesh=mesh,
        out_type=jax.ShapeDtypeStruct((_B, _N), jnp.float32),
        scratch_types=[
            pltpu.VMEM((_N,), jnp.float32),
            pltpu.VMEM((_N,), jnp.float32),
            pltpu.VMEM((_N + _L,), jnp.float32),
            pltpu.SemaphoreType.DMA,
            pltpu.SemaphoreType.DMA,
            pltpu.SemaphoreType.DMA,
            pltpu.SemaphoreType.DMA,
        ],
        compiler_params=pltpu.CompilerParams(needs_layout_passes=False),
    )
    def run(x_hbm, out_hbm, row_a, row_b, cand_v, si_a, si_b, so_a, so_b):
        wid = lax.axis_index("s") * 2 + lax.axis_index("c")
        base = wid * _ROWS_PER_W
        bufs = (row_a, row_b)
        sin = (si_a, si_b)
        sout = (so_a, so_b)

        def cp_in(r, b):
            return pltpu.make_async_copy(x_hbm.at[base + r], bufs[b], sin[b])

        def cp_out(r, b):
            return pltpu.make_async_copy(bufs[b], out_hbm.at[base + r], sout[b])

        cp_in(0, 0).start()
        cp_in(1, 1).start()
        for r in range(_ROWS_PER_W):
            b = r % 2
            cp_in(r, b).wait()
            if r >= 1 and r + 1 < _ROWS_PER_W:
                cp_out(r - 1, 1 - b).wait()
                cp_in(r + 1, 1 - b).start()
            _row_sparsemax(bufs[b], cand_v)
            cp_out(r, b).start()
        cp_out(_ROWS_PER_W - 2, 0 if _ROWS_PER_W % 2 == 0 else 1).wait()
        cp_out(_ROWS_PER_W - 1, 1 if _ROWS_PER_W % 2 == 0 else 0).wait()

    return run(inputs)


# speedup vs baseline: 3.1169x; 1.0958x over previous
import functools

import jax
import jax.numpy as jnp
from jax import lax
from jax.experimental import pallas as pl
from jax.experimental.pallas import tpu as pltpu
from jax.experimental.pallas import tpu_sc as plsc

_B = 128
_N = 32768
_L = 16
_NVEC = _N // _L
_NW = 32
_ROWS_PER_W = _B // _NW
_BISECT_ITERS = 22
_UNROLL = 8


def _row_sparsemax(row_v, cand_v):
    iota = lax.iota(jnp.int32, _L)

    @plsc.parallel_loop(
        0, _NVEC, unroll=_UNROLL,
        carry=(jnp.full((_L,), -3e38, jnp.float32), jnp.int32(0)),
    )
    def filt(i, carry):
        m_run, w = carry
        v = row_v[pl.ds(i * _L, _L)]
        m_run = jnp.maximum(m_run, v)
        mask = v >= (m_run - 1.0)
        plsc.store_compressed(cand_v.at[pl.ds(w, _L)], v, mask=mask)
        pc = plsc.all_reduce_population_count(mask)
        return m_run, w + pc[0]

    m_run, w = filt
    m = jnp.max(m_run)
    plsc.store_scatter(cand_v, [w + iota], jnp.full((_L,), m - 2.0))
    nvec = w // _L + 1

    def bisect(_, carry):
        lo, hi = carry
        t = 0.5 * (lo + hi)

        def acc_fn(j, acc):
            v = cand_v[pl.ds(j * _L, _L)]
            return acc + jnp.maximum(v - t, 0.0)

        acc = lax.fori_loop(0, nvec, acc_fn, jnp.zeros((_L,), jnp.float32))
        s = jnp.sum(acc)
        ge = s >= 1.0
        return jnp.where(ge, t, lo), jnp.where(ge, hi, t)

    lo, _ = lax.fori_loop(0, _BISECT_ITERS, bisect, (m - 1.0, m))

    def ks_fn(j, carry):
        ak, asum = carry
        v = cand_v[pl.ds(j * _L, _L)]
        above = v > lo
        ak = ak + above.astype(jnp.float32)
        asum = asum + jnp.where(above, v, 0.0)
        return ak, asum

    ak, asum = lax.fori_loop(
        0, nvec, ks_fn,
        (jnp.zeros((_L,), jnp.float32), jnp.zeros((_L,), jnp.float32)),
    )
    tau = (jnp.full((_L,), jnp.sum(asum)) - 1.0) / jnp.full((_L,), jnp.sum(ak))

    @plsc.parallel_loop(0, _NVEC, unroll=_UNROLL)
    def out_loop(i):
        v = row_v[pl.ds(i * _L, _L)]
        row_v[pl.ds(i * _L, _L)] = jnp.maximum(v - tau, 0.0)


def kernel(inputs):
    mesh = plsc.VectorSubcoreMesh(core_axis_name="c", subcore_axis_name="s")

    @functools.partial(
        pl.kernel,
        mesh=mesh,
        out_type=jax.ShapeDtypeStruct((_B, _N), jnp.float32),
        scratch_types=[
            pltpu.VMEM((_N,), jnp.float32),
            pltpu.VMEM((_N,), jnp.float32),
            pltpu.VMEM((_N + _L,), jnp.float32),
            pltpu.SemaphoreType.DMA,
            pltpu.SemaphoreType.DMA,
            pltpu.SemaphoreType.DMA,
            pltpu.SemaphoreType.DMA,
        ],
        compiler_params=pltpu.CompilerParams(needs_layout_passes=False),
    )
    def run(x_hbm, out_hbm, row_a, row_b, cand_v, si_a, si_b, so_a, so_b):
        wid = lax.axis_index("s") * 2 + lax.axis_index("c")
        base = wid * _ROWS_PER_W
        bufs = (row_a, row_b)
        sin = (si_a, si_b)
        sout = (so_a, so_b)

        def cp_in(r, b):
            return pltpu.make_async_copy(x_hbm.at[base + r], bufs[b], sin[b])

        def cp_out(r, b):
            return pltpu.make_async_copy(bufs[b], out_hbm.at[base + r], sout[b])

        cp_in(0, 0).start()
        cp_in(1, 1).start()
        for r in range(_ROWS_PER_W):
            b = r % 2
            cp_in(r, b).wait()
            if r >= 1 and r + 1 < _ROWS_PER_W:
                cp_out(r - 1, 1 - b).wait()
                cp_in(r + 1, 1 - b).start()
            _row_sparsemax(bufs[b], cand_v)
            cp_out(r, b).start()
        cp_out(_ROWS_PER_W - 2, 0 if _ROWS_PER_W % 2 == 0 else 1).wait()
        cp_out(_ROWS_PER_W - 1, 1 if _ROWS_PER_W % 2 == 0 else 0).wait()

    return run(inputs)
